# baseline (speedup 1.0000x reference)
import jax
import jax.numpy as jnp
from jax.experimental import pallas as pl
from jax.experimental.pallas import tpu as pltpu


def _pick_bm(n, target):
    best = 8
    for bm in range(8, target + 1, 8):
        if n % bm == 0:
            best = bm
    return best


def _support_kernel(x_ref, w_ref, o_ref):
    xb = x_ref[...].astype(jnp.bfloat16)
    wb = w_ref[...].astype(jnp.bfloat16)
    o_ref[...] = jnp.dot(xb, wb, preferred_element_type=jnp.float32).astype(
        jnp.bfloat16
    )


def _agg_relu_kernel(adj_ref, s_ref, b_ref, o_ref):
    a = adj_ref[...].astype(jnp.bfloat16)
    acc = jnp.dot(a, s_ref[...], preferred_element_type=jnp.float32)
    o_ref[...] = jnp.maximum(acc + b_ref[...], 0.0).astype(jnp.bfloat16)


def _agg_logsoftmax_kernel(adj_ref, s_ref, b_ref, o_ref):
    a = adj_ref[...].astype(jnp.bfloat16)
    logits = jnp.dot(a, s_ref[...], preferred_element_type=jnp.float32)
    logits = logits + b_ref[...]
    m = jnp.max(logits, axis=1, keepdims=True)
    e = logits - m
    lse = jnp.log(jnp.sum(jnp.exp(e), axis=1, keepdims=True))
    o_ref[...] = e - lse


def _small_gemm(x, w):
    n, f = x.shape
    fo = w.shape[1]
    bm = _pick_bm(n, 2000)
    return pl.pallas_call(
        _support_kernel,
        grid=(n // bm,),
        in_specs=[
            pl.BlockSpec((bm, f), lambda i: (i, 0)),
            pl.BlockSpec((f, fo), lambda i: (0, 0)),
        ],
        out_specs=pl.BlockSpec((bm, fo), lambda i: (i, 0)),
        out_shape=jax.ShapeDtypeStruct((n, fo), jnp.bfloat16),
        compiler_params=pltpu.CompilerParams(
            dimension_semantics=("parallel",)
        ),
    )(x, w)


def _agg_pass(body, adj, s, b, out_dtype, bm):
    n = adj.shape[0]
    fo = s.shape[1]
    return pl.pallas_call(
        body,
        grid=(n // bm,),
        in_specs=[
            pl.BlockSpec((bm, n), lambda i: (i, 0)),
            pl.BlockSpec((n, fo), lambda i: (0, 0)),
            pl.BlockSpec((1, fo), lambda i: (0, 0)),
        ],
        out_specs=pl.BlockSpec((bm, fo), lambda i: (i, 0)),
        out_shape=jax.ShapeDtypeStruct((n, fo), out_dtype),
        compiler_params=pltpu.CompilerParams(
            dimension_semantics=("parallel",)
        ),
    )(adj, s, b)


def kernel(x, adj, W1, b1, W2, b2):
    n = adj.shape[0]
    nh = W1.shape[1]
    nc = W2.shape[1]
    bm = _pick_bm(n, 200)

    s1 = _small_gemm(x, W1)
    h = _agg_pass(
        _agg_relu_kernel, adj, s1, b1.reshape(1, nh), jnp.bfloat16, bm
    )
    s2 = _small_gemm(h, W2)
    out = _agg_pass(
        _agg_logsoftmax_kernel, adj, s2, b2.reshape(1, nc), jnp.float32, bm
    )
    return out


# speedup vs baseline: 1.0442x; 1.0442x over previous
import jax
import jax.numpy as jnp
from jax.experimental import pallas as pl
from jax.experimental.pallas import tpu as pltpu


def _pick_bm(n, target):
    best = 8
    for bm in range(8, target + 1, 8):
        if n % bm == 0:
            best = bm
    return best


def _make_fused_kernel(bm):
    def _fused(x_ref, adj_ref, w1_ref, b1_ref, w2_ref, b2_ref, o_ref,
               s1_ref, h_ref, s2_ref):
        p = pl.program_id(0)
        i = pl.program_id(1)

        @pl.when((p == 0) & (i == 0))
        def _():
            s1_ref[...] = jnp.dot(
                x_ref[...].astype(jnp.bfloat16),
                w1_ref[...].astype(jnp.bfloat16),
                preferred_element_type=jnp.float32,
            ).astype(jnp.bfloat16)

        @pl.when(p == 0)
        def _():
            a = adj_ref[...].astype(jnp.bfloat16)
            acc = jnp.dot(a, s1_ref[...], preferred_element_type=jnp.float32)
            h_ref[pl.ds(i * bm, bm), :] = jnp.maximum(
                acc + b1_ref[...], 0.0
            ).astype(jnp.bfloat16)

        @pl.when((p == 1) & (i == 0))
        def _():
            s2_ref[...] = jnp.dot(
                h_ref[...],
                w2_ref[...].astype(jnp.bfloat16),
                preferred_element_type=jnp.float32,
            ).astype(jnp.bfloat16)

        @pl.when(p == 1)
        def _():
            a = adj_ref[...].astype(jnp.bfloat16)
            logits = jnp.dot(a, s2_ref[...], preferred_element_type=jnp.float32)
            logits = logits + b2_ref[...]
            m = jnp.max(logits, axis=1, keepdims=True)
            e = logits - m
            o_ref[...] = e - jnp.log(jnp.sum(jnp.exp(e), axis=1, keepdims=True))

    return _fused


def kernel(x, adj, W1, b1, W2, b2):
    n, nf = x.shape
    nh = W1.shape[1]
    nc = W2.shape[1]
    bm = _pick_bm(n, 200)

    return pl.pallas_call(
        _make_fused_kernel(bm),
        grid=(2, n // bm),
        in_specs=[
            pl.BlockSpec((n, nf), lambda p, i: (0, 0)),
            pl.BlockSpec((bm, n), lambda p, i: (i, 0)),
            pl.BlockSpec((nf, nh), lambda p, i: (0, 0)),
            pl.BlockSpec((1, nh), lambda p, i: (0, 0)),
            pl.BlockSpec((nh, nc), lambda p, i: (0, 0)),
            pl.BlockSpec((1, nc), lambda p, i: (0, 0)),
        ],
        out_specs=pl.BlockSpec((bm, nc), lambda p, i: (i, 0)),
        out_shape=jax.ShapeDtypeStruct((n, nc), jnp.float32),
        scratch_shapes=[
            pltpu.VMEM((n, nh), jnp.bfloat16),
            pltpu.VMEM((n, nh), jnp.bfloat16),
            pltpu.VMEM((n, nc), jnp.bfloat16),
        ],
        compiler_params=pltpu.CompilerParams(
            dimension_semantics=("arbitrary", "arbitrary")
        ),
    )(x, adj, W1, b1.reshape(1, nh), W2, b2.reshape(1, nc))


# speedup vs baseline: 1.0891x; 1.0430x over previous
import jax
import jax.numpy as jnp
from jax.experimental import pallas as pl
from jax.experimental.pallas import tpu as pltpu


def _pick_bm(n, target):
    best = 8
    for bm in range(8, target + 1, 8):
        if n % bm == 0:
            best = bm
    return best


def _make_fused_kernel(bm):
    def _fused(x_ref, adj_ref, w1_ref, b1_ref, w2_ref, b2_ref, o_ref,
               s1_ref, h_ref, s2_ref):
        p = pl.program_id(0)
        i = pl.program_id(1)

        @pl.when((p == 0) & (i == 0))
        def _():
            s1_ref[...] = jnp.dot(
                x_ref[...].astype(jnp.bfloat16),
                w1_ref[...].astype(jnp.bfloat16),
                preferred_element_type=jnp.float32,
            ).astype(jnp.bfloat16)

        @pl.when(p == 0)
        def _():
            a = adj_ref[...].astype(jnp.bfloat16)
            acc = jnp.dot(a, s1_ref[...], preferred_element_type=jnp.float32)
            h_ref[pl.ds(i * bm, bm), :] = jnp.maximum(
                acc + b1_ref[...], 0.0
            ).astype(jnp.bfloat16)

        @pl.when((p == 1) & (i == 0))
        def _():
            s2_ref[...] = jnp.dot(
                h_ref[...],
                w2_ref[...].astype(jnp.bfloat16),
                preferred_element_type=jnp.float32,
            ).astype(jnp.bfloat16)

        @pl.when(p == 1)
        def _():
            a = adj_ref[...].astype(jnp.bfloat16)
            logits = jnp.dot(a, s2_ref[...], preferred_element_type=jnp.float32)
            logits = logits + b2_ref[...]
            m = jnp.max(logits, axis=1, keepdims=True)
            e = logits - m
            o_ref[...] = e - jnp.log(jnp.sum(jnp.exp(e), axis=1, keepdims=True))

    return _fused


def kernel(x, adj, W1, b1, W2, b2):
    n, nf = x.shape
    nh = W1.shape[1]
    nc = W2.shape[1]
    bm = _pick_bm(n, 400)

    return pl.pallas_call(
        _make_fused_kernel(bm),
        grid=(2, n // bm),
        in_specs=[
            pl.BlockSpec((n, nf), lambda p, i: (0, 0)),
            pl.BlockSpec((bm, n), lambda p, i: (i, 0)),
            pl.BlockSpec((nf, nh), lambda p, i: (0, 0)),
            pl.BlockSpec((1, nh), lambda p, i: (0, 0)),
            pl.BlockSpec((nh, nc), lambda p, i: (0, 0)),
            pl.BlockSpec((1, nc), lambda p, i: (0, 0)),
        ],
        out_specs=pl.BlockSpec((bm, nc), lambda p, i: (i, 0)),
        out_shape=jax.ShapeDtypeStruct((n, nc), jnp.float32),
        scratch_shapes=[
            pltpu.VMEM((n, nh), jnp.bfloat16),
            pltpu.VMEM((n, nh), jnp.bfloat16),
            pltpu.VMEM((n, nc), jnp.bfloat16),
        ],
        compiler_params=pltpu.CompilerParams(
            dimension_semantics=("arbitrary", "arbitrary")
        ),
    )(x, adj, W1, b1.reshape(1, nh), W2, b2.reshape(1, nc))


# speedup vs baseline: 1.0892x; 1.0001x over previous
import jax
import jax.numpy as jnp
from jax.experimental import pallas as pl
from jax.experimental.pallas import tpu as pltpu

_DN = (((1,), (0,)), ((), ()))


def _pick_bm(n, target):
    best = 8
    for bm in range(8, target + 1, 8):
        if n % bm == 0:
            best = bm
    return best


def _dot(a, b):
    return jax.lax.dot_general(
        a, b, _DN,
        precision=jax.lax.Precision.DEFAULT,
        preferred_element_type=jnp.float32,
    )


def _make_fused_kernel(bm):
    def _fused(x_ref, adj_ref, w1_ref, b1_ref, w2_ref, b2_ref, o_ref,
               s1_ref, h_ref, s2_ref):
        p = pl.program_id(0)
        i = pl.program_id(1)

        @pl.when((p == 0) & (i == 0))
        def _():
            s1_ref[...] = _dot(x_ref[...], w1_ref[...])

        @pl.when(p == 0)
        def _():
            acc = _dot(adj_ref[...], s1_ref[...])
            h_ref[pl.ds(i * bm, bm), :] = jnp.maximum(acc + b1_ref[...], 0.0)

        @pl.when((p == 1) & (i == 0))
        def _():
            s2_ref[...] = _dot(h_ref[...], w2_ref[...])

        @pl.when(p == 1)
        def _():
            logits = _dot(adj_ref[...], s2_ref[...]) + b2_ref[...]
            m = jnp.max(logits, axis=1, keepdims=True)
            e = logits - m
            o_ref[...] = e - jnp.log(jnp.sum(jnp.exp(e), axis=1, keepdims=True))

    return _fused


def kernel(x, adj, W1, b1, W2, b2):
    n, nf = x.shape
    nh = W1.shape[1]
    nc = W2.shape[1]
    bm = _pick_bm(n, 400)

    return pl.pallas_call(
        _make_fused_kernel(bm),
        grid=(2, n // bm),
        in_specs=[
            pl.BlockSpec((n, nf), lambda p, i: (0, 0)),
            pl.BlockSpec((bm, n), lambda p, i: (i, 0)),
            pl.BlockSpec((nf, nh), lambda p, i: (0, 0)),
            pl.BlockSpec((1, nh), lambda p, i: (0, 0)),
            pl.BlockSpec((nh, nc), lambda p, i: (0, 0)),
            pl.BlockSpec((1, nc), lambda p, i: (0, 0)),
        ],
        out_specs=pl.BlockSpec((bm, nc), lambda p, i: (i, 0)),
        out_shape=jax.ShapeDtypeStruct((n, nc), jnp.float32),
        scratch_shapes=[
            pltpu.VMEM((n, nh), jnp.float32),
            pltpu.VMEM((n, nh), jnp.float32),
            pltpu.VMEM((n, nc), jnp.float32),
        ],
        compiler_params=pltpu.CompilerParams(
            dimension_semantics=("arbitrary", "arbitrary")
        ),
    )(x, adj, W1, b1.reshape(1, nh), W2, b2.reshape(1, nc))


# speedup vs baseline: 1.0925x; 1.0031x over previous
import jax
import jax.numpy as jnp
from jax.experimental import pallas as pl
from jax.experimental.pallas import tpu as pltpu

_DN = (((1,), (0,)), ((), ()))


def _pick_bm(n, target):
    best = 8
    for bm in range(8, target + 1, 8):
        if n % bm == 0:
            best = bm
    return best


def _dot(a, b):
    return jax.lax.dot_general(
        a, b, _DN,
        precision=jax.lax.Precision.DEFAULT,
        preferred_element_type=jnp.float32,
    )


def _make_fused_kernel(bm):
    def _fused(x_ref, adj_ref, w1_ref, b1_ref, w2_ref, b2_ref, o_ref,
               s1_ref, h_ref, s2_ref):
        p = pl.program_id(0)
        i = pl.program_id(1)

        @pl.when((p == 0) & (i == 0))
        def _():
            s1_ref[...] = _dot(x_ref[...], w1_ref[...])

        @pl.when(p == 0)
        def _():
            acc = _dot(adj_ref[...], s1_ref[...])
            h_ref[pl.ds(i * bm, bm), :] = jnp.maximum(acc + b1_ref[...], 0.0)

        @pl.when((p == 1) & (i == 0))
        def _():
            s2_ref[...] = _dot(h_ref[...], w2_ref[...])

        @pl.when(p == 1)
        def _():
            logits = _dot(adj_ref[...], s2_ref[...]) + b2_ref[...]
            m = jnp.max(logits, axis=1, keepdims=True)
            e = logits - m
            o_ref[...] = e - jnp.log(jnp.sum(jnp.exp(e), axis=1, keepdims=True))

    return _fused


def kernel(x, adj, W1, b1, W2, b2):
    n, nf = x.shape
    nh = W1.shape[1]
    nc = W2.shape[1]
    bm = _pick_bm(n, 400)

    return pl.pallas_call(
        _make_fused_kernel(bm),
        grid=(2, n // bm),
        in_specs=[
            pl.BlockSpec((n, nf), lambda p, i: (0, 0)),
            pl.BlockSpec((bm, n), lambda p, i: (i, 0)),
            pl.BlockSpec((nf, nh), lambda p, i: (0, 0)),
            pl.BlockSpec((1, nh), lambda p, i: (0, 0)),
            pl.BlockSpec((nh, nc), lambda p, i: (0, 0)),
            pl.BlockSpec((1, nc), lambda p, i: (0, 0)),
        ],
        out_specs=pl.BlockSpec((bm, nc), lambda p, i: (p * i, 0)),
        out_shape=jax.ShapeDtypeStruct((n, nc), jnp.float32),
        scratch_shapes=[
            pltpu.VMEM((n, nh), jnp.float32),
            pltpu.VMEM((n, nh), jnp.float32),
            pltpu.VMEM((n, nc), jnp.float32),
        ],
        compiler_params=pltpu.CompilerParams(
            dimension_semantics=("arbitrary", "arbitrary")
        ),
    )(x, adj, W1, b1.reshape(1, nh), W2, b2.reshape(1, nc))
